# baseline (speedup 1.0000x reference)
import jax
import jax.numpy as jnp
from jax.experimental import pallas as pl
from jax.experimental.pallas import tpu as pltpu

B, D, U, M = 1024, 128, 64, 16384
USAGE_DECAY = 0.95
MB = 256
NB = M // MB


def _dg(a, b, ca, cb):
    return jax.lax.dot_general(a, b, (((ca,), (cb,)), ((), ())),
                               preferred_element_type=jnp.float32)


def _body(x_ref, rT_ref, hT_ref, cT_ref, wk_ref, uk_ref, bk_ref, wg_ref,
          mT_ref, cwu1_ref, cwlu1_ref, cwr1_ref,
          keyT_ref, cnewT_ref, readT_ref, cwr_ref, cwu_ref, cwlu_ref,
          memT_ref,
          nkey_s, key_s, memwT_s, lmask_s, blkmin_s, cmin_s, sel_s):
    s = pl.program_id(0)

    @pl.when(s == 0)
    def _lstm():
        z = (_dg(x_ref[...], wk_ref[0:D, :], 1, 0)
             + _dg(rT_ref[...], wk_ref[D:D + U, :], 0, 0)
             + _dg(hT_ref[...], uk_ref[...], 0, 0)
             + bk_ref[0:1, :])
        eye = (jax.lax.broadcasted_iota(jnp.int32, (U, U), 0)
               == jax.lax.broadcasted_iota(jnp.int32, (U, U), 1)
               ).astype(jnp.float32)
        c = _dg(cT_ref[...], eye, 0, 0)
        gi = jax.nn.sigmoid(z[:, 0 * U:1 * U])
        gf = jax.nn.sigmoid(z[:, 1 * U:2 * U])
        c_new = gf * c + gi * jnp.tanh(z[:, 2 * U:3 * U])
        go = jax.nn.sigmoid(z[:, 3 * U:4 * U])
        key = go * jnp.tanh(c_new)
        keyT_ref[...] = _dg(eye, key, 1, 1)
        cnewT_ref[...] = _dg(eye, c_new, 1, 1)
        key_s[...] = key
        nkey_s[...] = key / jnp.sqrt(
            jnp.maximum(jnp.sum(key * key, axis=1, keepdims=True), 1e-12))

    @pl.when(s < NB)
    def _phase0():
        i = s
        mT = mT_ref[...]
        nmT = mT / jnp.sqrt(
            jnp.maximum(jnp.sum(mT * mT, axis=0, keepdims=True), 1e-12))
        cos = _dg(nmT, nkey_s[...], 0, 1)
        e = jnp.exp(cos)
        cwr = e / jnp.sum(e, axis=1, keepdims=True)
        cwr_ref[...] = cwr

        wg = wg_ref[0, 0]
        cww = wg * cwr1_ref[...] + (1.0 - wg) + cwlu1_ref[...]
        cwu = USAGE_DECAY * cwu1_ref[...] + cwr + cww
        cwu_ref[...] = cwu

        memwT_s[i] = _dg(key_s[...], cww, 0, 1)
        rT_part = _dg(mT, cwr, 1, 0)

        blkmin = jnp.min(cwu, axis=0, keepdims=True)
        lmask_s[pl.ds(i * MB, MB), :] = (cwu == blkmin).astype(jnp.int8)
        blkmin_s[pl.ds(i, 1), :] = blkmin

        @pl.when(i == 0)
        def _init():
            readT_ref[...] = rT_part

        @pl.when(i != 0)
        def _acc():
            readT_ref[...] += rT_part

    @pl.when(s >= NB)
    def _phase1():
        i = s - NB

        @pl.when(s == NB)
        def _select():
            bm = blkmin_s[...]
            cm = jnp.min(bm, axis=0, keepdims=True)
            cmin_s[...] = cm
            minv = jnp.min(cm)
            lane = jax.lax.broadcasted_iota(jnp.int32, (1, B), 1)
            i_nth = jnp.min(jnp.where(cm == minv, lane, 2 ** 30))
            colf = (lane == i_nth).astype(jnp.float32)
            blks = jax.lax.broadcasted_iota(jnp.int32, (NB, B), 0)
            hit = (bm == cm).astype(jnp.float32) * colf
            bsel = jnp.max(jnp.where(hit > 0.0, blks, -1))
            lblk = lmask_s[pl.ds(bsel * MB, MB), :].astype(jnp.float32)
            rows = jax.lax.broadcasted_iota(jnp.int32, (MB, B), 0)
            rsel = jnp.max(jnp.where(lblk * colf > 0.0, rows, -1))
            sel_s[0] = bsel * MB + rsel

        lmin = lmask_s[pl.ds(i * MB, MB), :].astype(jnp.float32)
        gmin = (blkmin_s[pl.ds(i, 1), :] == cmin_s[...]).astype(jnp.float32)
        cwlu_ref[...] = lmin * gmin

        sel = sel_s[0]
        lanes = jax.lax.broadcasted_iota(jnp.int32, (1, MB), 1) + i * MB
        keep = (lanes != sel).astype(jnp.float32)
        memT_ref[...] = memwT_s[i] + (keep * float(B)) * mT_ref[...]


def kernel(inputs, r_tm1, m_tm1, c_wu_tm1, c_wlu_tm1, c_wr_tm1, h_tm1,
           c_tm1, write_gate, Wk, Uk, bk):
    bk8 = jnp.broadcast_to(bk.reshape(1, 4 * U), (8, 4 * U))
    wg8 = jnp.broadcast_to(jax.nn.sigmoid(write_gate).reshape(1, 1), (8, 128))
    f32 = jnp.float32

    mT = m_tm1.T
    rT = r_tm1.T
    hT = h_tm1.T
    cT = c_tm1.T

    fixed = lambda s: (0, 0)
    blk_p0 = lambda s: (jnp.minimum(s, NB - 1), 0)
    blk_p1 = lambda s: (jnp.maximum(s - NB, 0), 0)
    lane_both = lambda s: (0, jnp.where(s < NB, s, s - NB))
    lane_p1 = lambda s: (0, jnp.maximum(s - NB, 0))

    (keyT, cnewT, readT, c_wr, c_wu, c_wlu, memT) = pl.pallas_call(
        _body,
        grid=(2 * NB,),
        in_specs=[
            pl.BlockSpec((B, D), fixed),
            pl.BlockSpec((U, B), fixed),
            pl.BlockSpec((U, B), fixed),
            pl.BlockSpec((U, B), fixed),
            pl.BlockSpec((D + U, 4 * U), fixed),
            pl.BlockSpec((U, 4 * U), fixed),
            pl.BlockSpec((8, 4 * U), fixed),
            pl.BlockSpec((8, 128), fixed),
            pl.BlockSpec((U, MB), lane_both),
            pl.BlockSpec((MB, B), blk_p0),
            pl.BlockSpec((MB, B), blk_p0),
            pl.BlockSpec((MB, B), blk_p0),
        ],
        out_specs=[
            pl.BlockSpec((U, B), fixed),
            pl.BlockSpec((U, B), fixed),
            pl.BlockSpec((U, B), fixed),
            pl.BlockSpec((MB, B), blk_p0),
            pl.BlockSpec((MB, B), blk_p0),
            pl.BlockSpec((MB, B), blk_p1),
            pl.BlockSpec((U, MB), lane_p1),
        ],
        out_shape=[
            jax.ShapeDtypeStruct((U, B), f32),
            jax.ShapeDtypeStruct((U, B), f32),
            jax.ShapeDtypeStruct((U, B), f32),
            jax.ShapeDtypeStruct((M, B), f32),
            jax.ShapeDtypeStruct((M, B), f32),
            jax.ShapeDtypeStruct((M, B), f32),
            jax.ShapeDtypeStruct((U, M), f32),
        ],
        scratch_shapes=[
            pltpu.VMEM((B, U), f32),
            pltpu.VMEM((B, U), f32),
            pltpu.VMEM((NB, U, MB), f32),
            pltpu.VMEM((M, B), jnp.int8),
            pltpu.VMEM((NB, B), f32),
            pltpu.VMEM((1, B), f32),
            pltpu.SMEM((1,), jnp.int32),
        ],
    )(inputs, rT, hT, cT, Wk, Uk, bk8, wg8, mT, c_wu_tm1, c_wlu_tm1,
      c_wr_tm1)

    read = readT.T
    return (read, read, memT.T, c_wu, c_wlu, c_wr, keyT.T, cnewT.T)


# speedup vs baseline: 1.2073x; 1.2073x over previous
import jax
import jax.numpy as jnp
from jax.experimental import pallas as pl
from jax.experimental.pallas import tpu as pltpu

B, D, U, M = 1024, 128, 64, 16384
USAGE_DECAY = 0.95
MB = 512
NB = M // MB


def _dg(a, b, ca, cb):
    return jax.lax.dot_general(a, b, (((ca,), (cb,)), ((), ())),
                               preferred_element_type=jnp.float32)


def _body(x_ref, rT_ref, hT_ref, cT_ref, wk_ref, uk_ref, bk_ref, wg_ref,
          mT_ref, cwu1_ref, cwlu1_ref, cwr1_ref,
          keyT_ref, cnewT_ref, readT_ref, cwr_ref, cwu_ref, cwlu_ref,
          memT_ref,
          nkey_s, key_s, memwT_s, lmask_s, blkmin_s, cmin_s, sel_s):
    s = pl.program_id(0)

    @pl.when(s == 0)
    def _lstm():
        z = (_dg(x_ref[...], wk_ref[0:D, :], 1, 0)
             + _dg(rT_ref[...], wk_ref[D:D + U, :], 0, 0)
             + _dg(hT_ref[...], uk_ref[...], 0, 0)
             + bk_ref[0:1, :])
        eye = (jax.lax.broadcasted_iota(jnp.int32, (U, U), 0)
               == jax.lax.broadcasted_iota(jnp.int32, (U, U), 1)
               ).astype(jnp.float32)
        c = _dg(cT_ref[...], eye, 0, 0)
        gi = jax.nn.sigmoid(z[:, 0 * U:1 * U])
        gf = jax.nn.sigmoid(z[:, 1 * U:2 * U])
        c_new = gf * c + gi * jnp.tanh(z[:, 2 * U:3 * U])
        go = jax.nn.sigmoid(z[:, 3 * U:4 * U])
        key = go * jnp.tanh(c_new)
        keyT_ref[...] = _dg(eye, key, 1, 1)
        cnewT_ref[...] = _dg(eye, c_new, 1, 1)
        key_s[...] = key
        nkey_s[...] = key / jnp.sqrt(
            jnp.maximum(jnp.sum(key * key, axis=1, keepdims=True), 1e-12))

    @pl.when(s < NB)
    def _phase0():
        i = s
        mT = mT_ref[...]
        nmT = mT / jnp.sqrt(
            jnp.maximum(jnp.sum(mT * mT, axis=0, keepdims=True), 1e-12))
        cos = _dg(nmT, nkey_s[...], 0, 1)
        e = jnp.exp(cos)
        cwr = e / jnp.sum(e, axis=1, keepdims=True)
        cwr_ref[...] = cwr

        wg = wg_ref[0, 0]
        cww = wg * cwr1_ref[...] + (1.0 - wg) + cwlu1_ref[...]
        cwu = USAGE_DECAY * cwu1_ref[...] + cwr + cww
        cwu_ref[...] = cwu

        memwT_s[i] = _dg(key_s[...], cww, 0, 1)
        rT_part = _dg(mT, cwr, 1, 0)

        blkmin = jnp.min(cwu, axis=0, keepdims=True)
        lmask_s[pl.ds(i * MB, MB), :] = (cwu == blkmin).astype(jnp.int8)
        blkmin_s[pl.ds(i, 1), :] = blkmin

        @pl.when(i == 0)
        def _init():
            readT_ref[...] = rT_part

        @pl.when(i != 0)
        def _acc():
            readT_ref[...] += rT_part

    @pl.when(s >= NB)
    def _phase1():
        i = s - NB

        @pl.when(s == NB)
        def _select():
            bm = blkmin_s[...]
            cm = jnp.min(bm, axis=0, keepdims=True)
            cmin_s[...] = cm
            minv = jnp.min(cm)
            lane = jax.lax.broadcasted_iota(jnp.int32, (1, B), 1)
            i_nth = jnp.min(jnp.where(cm == minv, lane, 2 ** 30))
            colf = (lane == i_nth).astype(jnp.float32)
            blks = jax.lax.broadcasted_iota(jnp.int32, (NB, B), 0)
            hit = (bm == cm).astype(jnp.float32) * colf
            bsel = jnp.max(jnp.where(hit > 0.0, blks, -1))
            lblk = lmask_s[pl.ds(bsel * MB, MB), :].astype(jnp.float32)
            rows = jax.lax.broadcasted_iota(jnp.int32, (MB, B), 0)
            rsel = jnp.max(jnp.where(lblk * colf > 0.0, rows, -1))
            sel_s[0] = bsel * MB + rsel

        lmin = lmask_s[pl.ds(i * MB, MB), :].astype(jnp.float32)
        gmin = (blkmin_s[pl.ds(i, 1), :] == cmin_s[...]).astype(jnp.float32)
        cwlu_ref[...] = lmin * gmin

        sel = sel_s[0]
        lanes = jax.lax.broadcasted_iota(jnp.int32, (1, MB), 1) + i * MB
        keep = (lanes != sel).astype(jnp.float32)
        memT_ref[...] = memwT_s[i] + (keep * float(B)) * mT_ref[...]


def kernel(inputs, r_tm1, m_tm1, c_wu_tm1, c_wlu_tm1, c_wr_tm1, h_tm1,
           c_tm1, write_gate, Wk, Uk, bk):
    bk8 = jnp.broadcast_to(bk.reshape(1, 4 * U), (8, 4 * U))
    wg8 = jnp.broadcast_to(jax.nn.sigmoid(write_gate).reshape(1, 1), (8, 128))
    f32 = jnp.float32

    mT = m_tm1.T
    rT = r_tm1.T
    hT = h_tm1.T
    cT = c_tm1.T

    fixed = lambda s: (0, 0)
    blk_p0 = lambda s: (jnp.minimum(s, NB - 1), 0)
    blk_p1 = lambda s: (jnp.maximum(s - NB, 0), 0)
    lane_both = lambda s: (0, jnp.where(s < NB, s, s - NB))
    lane_p1 = lambda s: (0, jnp.maximum(s - NB, 0))

    (keyT, cnewT, readT, c_wr, c_wu, c_wlu, memT) = pl.pallas_call(
        _body,
        grid=(2 * NB,),
        in_specs=[
            pl.BlockSpec((B, D), fixed),
            pl.BlockSpec((U, B), fixed),
            pl.BlockSpec((U, B), fixed),
            pl.BlockSpec((U, B), fixed),
            pl.BlockSpec((D + U, 4 * U), fixed),
            pl.BlockSpec((U, 4 * U), fixed),
            pl.BlockSpec((8, 4 * U), fixed),
            pl.BlockSpec((8, 128), fixed),
            pl.BlockSpec((U, MB), lane_both),
            pl.BlockSpec((MB, B), blk_p0),
            pl.BlockSpec((MB, B), blk_p0),
            pl.BlockSpec((MB, B), blk_p0),
        ],
        out_specs=[
            pl.BlockSpec((U, B), fixed),
            pl.BlockSpec((U, B), fixed),
            pl.BlockSpec((U, B), fixed),
            pl.BlockSpec((MB, B), blk_p0),
            pl.BlockSpec((MB, B), blk_p0),
            pl.BlockSpec((MB, B), blk_p1),
            pl.BlockSpec((U, MB), lane_p1),
        ],
        out_shape=[
            jax.ShapeDtypeStruct((U, B), f32),
            jax.ShapeDtypeStruct((U, B), f32),
            jax.ShapeDtypeStruct((U, B), f32),
            jax.ShapeDtypeStruct((M, B), f32),
            jax.ShapeDtypeStruct((M, B), f32),
            jax.ShapeDtypeStruct((M, B), f32),
            jax.ShapeDtypeStruct((U, M), f32),
        ],
        scratch_shapes=[
            pltpu.VMEM((B, U), f32),
            pltpu.VMEM((B, U), f32),
            pltpu.VMEM((NB, U, MB), f32),
            pltpu.VMEM((M, B), jnp.int8),
            pltpu.VMEM((NB, B), f32),
            pltpu.VMEM((1, B), f32),
            pltpu.SMEM((1,), jnp.int32),
        ],
    )(inputs, rT, hT, cT, Wk, Uk, bk8, wg8, mT, c_wu_tm1, c_wlu_tm1,
      c_wr_tm1)

    read = readT.T
    return (read, read, memT.T, c_wu, c_wlu, c_wr, keyT.T, cnewT.T)


# speedup vs baseline: 1.2638x; 1.0469x over previous
import jax
import jax.numpy as jnp
from jax.experimental import pallas as pl
from jax.experimental.pallas import tpu as pltpu

B, D, U, M = 1024, 128, 64, 16384
USAGE_DECAY = 0.95
MB = 512
NB = M // MB


def _dg(a, b, ca, cb):
    return jax.lax.dot_general(a, b, (((ca,), (cb,)), ((), ())),
                               preferred_element_type=jnp.float32)


def _body(x_ref, rT_ref, hT_ref, cT_ref, wk_ref, uk_ref, bk_ref, wg_ref,
          mT_ref, cwu1_ref, cwlu1_ref, cwr1_ref,
          keyT_ref, cnewT_ref, readT_ref, cwr_ref, cwu_ref, cwlu_ref,
          memT_ref,
          nkey_s, key_s, memwT_s, mT_s, lmask_s, blkmin_s, cmin_s, sel_s):
    s = pl.program_id(0)

    @pl.when(s == 0)
    def _lstm():
        z = (_dg(x_ref[...], wk_ref[0:D, :], 1, 0)
             + _dg(rT_ref[...], wk_ref[D:D + U, :], 0, 0)
             + _dg(hT_ref[...], uk_ref[...], 0, 0)
             + bk_ref[0:1, :])
        eye = (jax.lax.broadcasted_iota(jnp.int32, (U, U), 0)
               == jax.lax.broadcasted_iota(jnp.int32, (U, U), 1)
               ).astype(jnp.float32)
        c = _dg(cT_ref[...], eye, 0, 0)
        gi = jax.nn.sigmoid(z[:, 0 * U:1 * U])
        gf = jax.nn.sigmoid(z[:, 1 * U:2 * U])
        c_new = gf * c + gi * jnp.tanh(z[:, 2 * U:3 * U])
        go = jax.nn.sigmoid(z[:, 3 * U:4 * U])
        key = go * jnp.tanh(c_new)
        keyT_ref[...] = _dg(eye, key, 1, 1)
        cnewT_ref[...] = _dg(eye, c_new, 1, 1)
        key_s[...] = key
        nkey_s[...] = key / jnp.sqrt(
            jnp.maximum(jnp.sum(key * key, axis=1, keepdims=True), 1e-12))

    @pl.when(s < NB)
    def _phase0():
        i = s
        mT = mT_ref[...]
        nmT = mT / jnp.sqrt(
            jnp.maximum(jnp.sum(mT * mT, axis=0, keepdims=True), 1e-12))
        cos = _dg(nmT, nkey_s[...], 0, 1)
        e = jnp.exp(cos)
        cwr = e / jnp.sum(e, axis=1, keepdims=True)
        cwr_ref[...] = cwr

        wg = wg_ref[0, 0]
        cww = wg * cwr1_ref[...] + (1.0 - wg) + cwlu1_ref[...]
        cwu = USAGE_DECAY * cwu1_ref[...] + cwr + cww
        cwu_ref[...] = cwu

        memwT_s[i] = _dg(key_s[...], cww, 0, 1)
        mT_s[i] = mT
        rT_part = _dg(mT, cwr, 1, 0)

        blkmin = jnp.min(cwu, axis=0, keepdims=True)
        lmask_s[pl.ds(i * MB, MB), :] = (cwu == blkmin).astype(jnp.int8)
        blkmin_s[pl.ds(i, 1), :] = blkmin

        @pl.when(i == 0)
        def _init():
            readT_ref[...] = rT_part

        @pl.when(i != 0)
        def _acc():
            readT_ref[...] += rT_part

    @pl.when(s >= NB)
    def _phase1():
        i = s - NB

        @pl.when(s == NB)
        def _select():
            bm = blkmin_s[...]
            cm = jnp.min(bm, axis=0, keepdims=True)
            cmin_s[...] = cm
            minv = jnp.min(cm)
            lane = jax.lax.broadcasted_iota(jnp.int32, (1, B), 1)
            i_nth = jnp.min(jnp.where(cm == minv, lane, 2 ** 30))
            colf = (lane == i_nth).astype(jnp.float32)
            blks = jax.lax.broadcasted_iota(jnp.int32, (NB, B), 0)
            hit = (bm == cm).astype(jnp.float32) * colf
            bsel = jnp.max(jnp.where(hit > 0.0, blks, -1))
            lblk = lmask_s[pl.ds(bsel * MB, MB), :].astype(jnp.float32)
            rows = jax.lax.broadcasted_iota(jnp.int32, (MB, B), 0)
            rsel = jnp.max(jnp.where(lblk * colf > 0.0, rows, -1))
            sel_s[0] = bsel * MB + rsel

        lmin = lmask_s[pl.ds(i * MB, MB), :].astype(jnp.float32)
        gmin = (blkmin_s[pl.ds(i, 1), :] == cmin_s[...]).astype(jnp.float32)
        cwlu_ref[...] = lmin * gmin

        sel = sel_s[0]
        lanes = jax.lax.broadcasted_iota(jnp.int32, (1, MB), 1) + i * MB
        keep = (lanes != sel).astype(jnp.float32)
        memT_ref[...] = memwT_s[i] + (keep * float(B)) * mT_s[i]


def kernel(inputs, r_tm1, m_tm1, c_wu_tm1, c_wlu_tm1, c_wr_tm1, h_tm1,
           c_tm1, write_gate, Wk, Uk, bk):
    bk8 = jnp.broadcast_to(bk.reshape(1, 4 * U), (8, 4 * U))
    wg8 = jnp.broadcast_to(jax.nn.sigmoid(write_gate).reshape(1, 1), (8, 128))
    f32 = jnp.float32

    mT = m_tm1.T
    rT = r_tm1.T
    hT = h_tm1.T
    cT = c_tm1.T

    fixed = lambda s: (0, 0)
    blk_p0 = lambda s: (jnp.minimum(s, NB - 1), 0)
    blk_p1 = lambda s: (jnp.maximum(s - NB, 0), 0)
    lane_p0 = lambda s: (0, jnp.minimum(s, NB - 1))
    lane_p1 = lambda s: (0, jnp.maximum(s - NB, 0))

    (keyT, cnewT, readT, c_wr, c_wu, c_wlu, memT) = pl.pallas_call(
        _body,
        grid=(2 * NB,),
        in_specs=[
            pl.BlockSpec((B, D), fixed),
            pl.BlockSpec((U, B), fixed),
            pl.BlockSpec((U, B), fixed),
            pl.BlockSpec((U, B), fixed),
            pl.BlockSpec((D + U, 4 * U), fixed),
            pl.BlockSpec((U, 4 * U), fixed),
            pl.BlockSpec((8, 4 * U), fixed),
            pl.BlockSpec((8, 128), fixed),
            pl.BlockSpec((U, MB), lane_p0),
            pl.BlockSpec((MB, B), blk_p0),
            pl.BlockSpec((MB, B), blk_p0),
            pl.BlockSpec((MB, B), blk_p0),
        ],
        out_specs=[
            pl.BlockSpec((U, B), fixed),
            pl.BlockSpec((U, B), fixed),
            pl.BlockSpec((U, B), fixed),
            pl.BlockSpec((MB, B), blk_p0),
            pl.BlockSpec((MB, B), blk_p0),
            pl.BlockSpec((MB, B), blk_p1),
            pl.BlockSpec((U, MB), lane_p1),
        ],
        out_shape=[
            jax.ShapeDtypeStruct((U, B), f32),
            jax.ShapeDtypeStruct((U, B), f32),
            jax.ShapeDtypeStruct((U, B), f32),
            jax.ShapeDtypeStruct((M, B), f32),
            jax.ShapeDtypeStruct((M, B), f32),
            jax.ShapeDtypeStruct((M, B), f32),
            jax.ShapeDtypeStruct((U, M), f32),
        ],
        scratch_shapes=[
            pltpu.VMEM((B, U), f32),
            pltpu.VMEM((B, U), f32),
            pltpu.VMEM((NB, U, MB), f32),
            pltpu.VMEM((NB, U, MB), f32),
            pltpu.VMEM((M, B), jnp.int8),
            pltpu.VMEM((NB, B), f32),
            pltpu.VMEM((1, B), f32),
            pltpu.SMEM((1,), jnp.int32),
        ],
    )(inputs, rT, hT, cT, Wk, Uk, bk8, wg8, mT, c_wu_tm1, c_wlu_tm1,
      c_wr_tm1)

    read = readT.T
    return (read, read, memT.T, c_wu, c_wlu, c_wr, keyT.T, cnewT.T)
